# baseline (speedup 1.0000x reference)
import functools

import jax
import jax.numpy as jnp
from jax import lax
from jax.experimental import pallas as pl
from jax.experimental.pallas import tpu as pltpu
from jax.experimental.pallas import tpu_sc as plsc

VOCAB = 1000000
D = 200
L = 200
B = 4096
NLANE = 16
NCHUNK = 13
CHUNK_OFFS = tuple(list(range(0, 192, 16)) + [184])
NW = 32
BPW = B // NW


def _sc_logits_kernel(idx_hbm, table_hbm, w_hbm, c0_hbm, out_hbm,
                      idx_v, rows_v, w_v, c0_v, log_v, sem, gsem):
  wid = lax.axis_index("s") * 2 + lax.axis_index("c")
  base = wid * BPW
  pltpu.sync_copy(w_hbm, w_v)
  pltpu.sync_copy(c0_hbm, c0_v)

  def per_elem(b_local, carry):
    b_abs = base + b_local
    pltpu.sync_copy(idx_hbm.at[b_abs], idx_v)
    cp0 = pltpu.async_copy(table_hbm.at[idx_v.at[pl.ds(0, 104)]],
                           rows_v.at[pl.ds(0, 104)], sem)
    cp1 = pltpu.async_copy(table_hbm.at[idx_v.at[pl.ds(104, 96)]],
                           rows_v.at[pl.ds(104, 96)], gsem)
    cp0.wait()
    cp1.wait()

    def per_t(t, accs):
      a0, a1 = accs
      for k in range(NCHUNK):
        off = CHUNK_OFFS[k]
        r = rows_v[t, pl.ds(off, NLANE)]
        a0 = a0 + r * w_v[0, t, k, :]
        a1 = a1 + r * w_v[1, t, k, :]
      return (a0, a1)

    z = jnp.zeros((NLANE,), jnp.float32)
    a0, a1 = lax.fori_loop(0, L, per_t, (z, z))

    c0acc = jnp.zeros((NLANE,), jnp.float32)
    c1acc = jnp.zeros((NLANE,), jnp.float32)
    for k in range(NCHUNK):
      off = CHUNK_OFFS[k]
      iv = idx_v[pl.ds(off, NLANE)]
      m = iv == 0
      c0acc = c0acc + jnp.where(m, c0_v[0, k, :], 0.0)
      c1acc = c1acc + jnp.where(m, c0_v[1, k, :], 0.0)

    log_v[b_local, 0, :] = a0 - c0acc
    log_v[b_local, 1, :] = a1 - c1acc
    return carry

  lax.fori_loop(0, BPW, per_elem, 0)
  pltpu.sync_copy(log_v, out_hbm.at[pl.ds(base, BPW)])


def _make_sc_logits():
  mesh = plsc.VectorSubcoreMesh(core_axis_name="c", subcore_axis_name="s")
  return functools.partial(
      pl.kernel,
      mesh=mesh,
      compiler_params=pltpu.CompilerParams(use_tc_tiling_on_sc=False),
      out_type=jax.ShapeDtypeStruct((B, 2, NLANE), jnp.float32),
      scratch_types=[
          pltpu.VMEM((L,), jnp.int32),
          pltpu.VMEM((L, D), jnp.float32),
          pltpu.VMEM((2, L, NCHUNK, NLANE), jnp.float32),
          pltpu.VMEM((2, NCHUNK, NLANE), jnp.float32),
          pltpu.VMEM((BPW, 2, NLANE), jnp.float32),
          pltpu.SemaphoreType.DMA,
          pltpu.SemaphoreType.DMA,
      ],
  )(_sc_logits_kernel)


_sc_logits = _make_sc_logits()


def _softmax_body(p_ref, b_ref, o_ref):
  x = jnp.sum(p_ref[...], axis=-1) + b_ref[...]
  m = jnp.max(x, axis=-1, keepdims=True)
  e = jnp.exp(x - m)
  o_ref[...] = (x - m) - jnp.log(jnp.sum(e, axis=-1, keepdims=True))


def _log_softmax(partials, b):
  return pl.pallas_call(
      _softmax_body,
      out_shape=jax.ShapeDtypeStruct((B, 2), jnp.float32),
  )(partials, b.reshape(1, 2))


def kernel(input, embedding, W, b):
  idx = input.astype(jnp.int32)
  Wr = W.reshape(2, L, D)
  main = Wr[:, :, :192].reshape(2, L, 12, NLANE)
  tail = jnp.concatenate(
      [jnp.zeros((2, L, 8), Wr.dtype), Wr[:, :, 192:]], axis=-1
  ).reshape(2, L, 1, NLANE)
  W4 = jnp.concatenate([main, tail], axis=2)

  cvec = jnp.einsum("d,ctd->ct", embedding[0], Wr)
  cmain = cvec[:, :192].reshape(2, 12, NLANE)
  ctail = jnp.concatenate(
      [jnp.zeros((2, 8), cvec.dtype), cvec[:, 192:]], axis=-1
  ).reshape(2, 1, NLANE)
  c0sc = jnp.concatenate([cmain, ctail], axis=1)

  partials = _sc_logits(idx, embedding, W4, c0sc)
  return _log_softmax(partials, b)
